# baseline (speedup 1.0000x reference)
import functools

import jax
import jax.numpy as jnp
from jax import lax
from jax.experimental import pallas as pl
from jax.experimental.pallas import tpu as pltpu
from jax.experimental.pallas import tpu_sc as plsc

N = 10000
D = 128
E = 320000

NC = 2
NS = 16
NW = NC * NS
PER_W = E // NW
EC = 104
PER_W_PAD = 10192
ECN = PER_W_PAD // EC
N_PAD = 10112
ROWS_PER_TILE = N_PAD // NS
DEG_CHUNK = 2000

_mesh = plsc.VectorSubcoreMesh(
    core_axis_name="c", subcore_axis_name="s", num_cores=NC, num_subcores=NS
)
_sc_params = pltpu.CompilerParams(needs_layout_passes=False)


@functools.partial(
    pl.kernel,
    out_type=jax.ShapeDtypeStruct((NW * N,), jnp.float32),
    mesh=_mesh,
    scratch_types=[
        pltpu.VMEM((N,), jnp.float32),
        pltpu.VMEM((DEG_CHUNK,), jnp.int32),
    ],
    compiler_params=_sc_params,
)
def _deg_kernel(dst_hbm, out_hbm, hist, idxv):
    cid = lax.axis_index("c")
    sid = lax.axis_index("s")
    wid = sid * NC + cid
    zeros16 = jnp.zeros((16,), jnp.float32)
    ones16 = jnp.ones((16,), jnp.float32)

    def zero_body(i, carry):
        hist[pl.ds(i * 16, 16)] = zeros16
        return carry

    lax.fori_loop(0, N // 16, zero_body, 0)

    def chunk_body(c, carry):
        pltpu.sync_copy(
            dst_hbm.at[pl.ds(wid * PER_W + c * DEG_CHUNK, DEG_CHUNK)], idxv
        )

        def inner(j, carry2):
            idx = idxv[pl.ds(j * 16, 16)]
            plsc.addupdate_scatter(hist, [idx], ones16)
            return carry2

        lax.fori_loop(0, DEG_CHUNK // 16, inner, 0)
        return carry

    lax.fori_loop(0, PER_W // DEG_CHUNK, chunk_body, 0)
    pltpu.sync_copy(hist, out_hbm.at[pl.ds(wid * N, N)])


@functools.partial(
    pl.kernel,
    out_type=jax.ShapeDtypeStruct((NC, N_PAD, D), jnp.float32),
    mesh=_mesh,
    scratch_types=[
        pltpu.VMEM((EC,), jnp.int32),
        pltpu.VMEM((EC,), jnp.int32),
        pltpu.VMEM((EC,), jnp.int32),
        pltpu.VMEM((EC,), jnp.int32),
        pltpu.VMEM((EC, D), jnp.float32),
        pltpu.VMEM((EC, D), jnp.float32),
        pltpu.VMEM_SHARED((N_PAD, D), jnp.float32),
        pltpu.SemaphoreType.DMA,
        pltpu.SemaphoreType.DMA,
        pltpu.SemaphoreType.DMA,
        pltpu.SemaphoreType.DMA,
        pltpu.SemaphoreType.DMA,
        pltpu.SemaphoreType.DMA,
    ],
    compiler_params=_sc_params,
)
def _edge_kernel(
    g_hbm, src_hbm, dst_hbm, zero_hbm, out_hbm,
    srcA, srcB, dstA, dstB, rows0, rows1, acc,
    sem0, sem1, semS0, semS1, semD0, semD1,
):
    cid = lax.axis_index("c")
    sid = lax.axis_index("s")
    wid = sid * NC + cid
    base_row = sid * ROWS_PER_TILE
    ebase = wid * PER_W_PAD

    pltpu.sync_copy(
        zero_hbm.at[pl.ds(base_row, ROWS_PER_TILE)],
        acc.at[pl.ds(base_row, ROWS_PER_TILE)],
    )
    plsc.subcore_barrier()

    pltpu.sync_copy(src_hbm.at[pl.ds(ebase, EC)], srcA)
    pltpu.async_copy(g_hbm.at[srcA], rows0, sem0)
    pltpu.sync_copy(src_hbm.at[pl.ds(ebase + EC, EC)], srcB)
    pltpu.async_copy(dst_hbm.at[pl.ds(ebase, EC)], dstA, semD0)

    def chunk_body(i, carry):
        k0 = 2 * i
        k1 = k0 + 1
        pltpu.make_async_copy(g_hbm.at[srcA], rows0, sem0).wait()
        pltpu.async_copy(g_hbm.at[srcB], rows1, sem1)

        @pl.when(k0 + 2 < ECN)
        def _():
            pltpu.async_copy(src_hbm.at[pl.ds(ebase + (k0 + 2) * EC, EC)], srcA, semS0)

        pltpu.make_async_copy(dst_hbm.at[pl.ds(ebase + k0 * EC, EC)], dstA, semD0).wait()
        pltpu.async_copy(dst_hbm.at[pl.ds(ebase + k1 * EC, EC)], dstB, semD1)
        pltpu.sync_copy(rows0, acc.at[dstA], add=True)
        pltpu.make_async_copy(g_hbm.at[srcB], rows1, sem1).wait()

        @pl.when(k0 + 2 < ECN)
        def _():
            pltpu.make_async_copy(
                src_hbm.at[pl.ds(ebase + (k0 + 2) * EC, EC)], srcA, semS0
            ).wait()
            pltpu.async_copy(g_hbm.at[srcA], rows0, sem0)

        @pl.when(k0 + 3 < ECN)
        def _():
            pltpu.async_copy(src_hbm.at[pl.ds(ebase + (k0 + 3) * EC, EC)], srcB, semS1)

        pltpu.make_async_copy(dst_hbm.at[pl.ds(ebase + k1 * EC, EC)], dstB, semD1).wait()
        pltpu.sync_copy(rows1, acc.at[dstB], add=True)

        @pl.when(k0 + 3 < ECN)
        def _():
            pltpu.make_async_copy(
                src_hbm.at[pl.ds(ebase + (k0 + 3) * EC, EC)], srcB, semS1
            ).wait()

        @pl.when(k0 + 2 < ECN)
        def _():
            pltpu.async_copy(dst_hbm.at[pl.ds(ebase + (k0 + 2) * EC, EC)], dstA, semD0)

        return carry

    lax.fori_loop(0, ECN // 2, chunk_body, 0)
    plsc.subcore_barrier()
    pltpu.sync_copy(
        acc.at[pl.ds(base_row, ROWS_PER_TILE)],
        out_hbm.at[cid, pl.ds(base_row, ROWS_PER_TILE)],
    )


R = 1024


def _dinv_from_hist(hist_blk):
    deg = jnp.sum(hist_blk, axis=0) + 1.0
    return lax.rsqrt(deg)


def _pre_body(hist_ref, x_ref, w1_ref, g1_ref):
    dinv = _dinv_from_hist(hist_ref[...])[:, None]
    h = jnp.dot(x_ref[...], w1_ref[...], preferred_element_type=jnp.float32)
    g1_ref[...] = h * dinv


def _mid_body(p_ref, g1_ref, hist_ref, b1_ref, w2_ref, g2_ref):
    dinv = _dinv_from_hist(hist_ref[...])[:, None]
    h1 = jnp.maximum(dinv * (p_ref[0] + p_ref[1] + g1_ref[...]) + b1_ref[...], 0.0)
    g2_ref[...] = (
        jnp.dot(h1, w2_ref[...], preferred_element_type=jnp.float32) * dinv
    )


def _post_body(q_ref, g2_ref, hist_ref, b2_ref, out_ref):
    dinv = _dinv_from_hist(hist_ref[...])[:, None]
    out_ref[...] = jnp.maximum(
        dinv * (q_ref[0] + q_ref[1] + g2_ref[...]) + b2_ref[...], 0.0
    )


_row_spec = pl.BlockSpec((R, D), lambda i: (i, 0))
_pair_spec = pl.BlockSpec((NC, R, D), lambda i: (0, i, 0))
_hist_spec = pl.BlockSpec((NW, R), lambda i: (0, i))
_w_spec = pl.BlockSpec((D, D), lambda i: (0, 0))
_b_spec = pl.BlockSpec((D,), lambda i: (0,))
_nd_shape = jax.ShapeDtypeStruct((N, D), jnp.float32)
_grid = (pl.cdiv(N, R),)

_pre_call = pl.pallas_call(
    _pre_body,
    grid=_grid,
    in_specs=[_hist_spec, _row_spec, _w_spec],
    out_specs=_row_spec,
    out_shape=_nd_shape,
)

_mid_call = pl.pallas_call(
    _mid_body,
    grid=_grid,
    in_specs=[_pair_spec, _row_spec, _hist_spec, _b_spec, _w_spec],
    out_specs=_row_spec,
    out_shape=_nd_shape,
)

_post_call = pl.pallas_call(
    _post_body,
    grid=_grid,
    in_specs=[_pair_spec, _row_spec, _hist_spec, _b_spec],
    out_specs=_row_spec,
    out_shape=_nd_shape,
)


def kernel(x, edge_index, W1, b1, W2, b2):
    pad = PER_W_PAD - PER_W
    src = jnp.reshape(
        jnp.pad(jnp.reshape(edge_index[0], (NW, PER_W)), ((0, 0), (0, pad))),
        (NW * PER_W_PAD,),
    )
    dst = edge_index[1]
    pad_rows = jnp.broadcast_to(
        (N + jnp.arange(NW, dtype=jnp.int32))[:, None], (NW, pad)
    )
    dst3 = jnp.reshape(
        jnp.concatenate([jnp.reshape(dst, (NW, PER_W)), pad_rows], axis=1),
        (NW * PER_W_PAD,),
    )
    hist = jnp.reshape(_deg_kernel(dst), (NW, N))
    zeros = jnp.zeros((N_PAD, D), jnp.float32)
    g1 = _pre_call(hist, x, W1)
    p = _edge_kernel(g1, src, dst3, zeros)
    g2 = _mid_call(p, g1, hist, b1, W2)
    q = _edge_kernel(g2, src, dst3, zeros)
    return _post_call(q, g2, hist, b2)


# speedup vs baseline: 1.8529x; 1.8529x over previous
import functools

import jax
import jax.numpy as jnp
from jax import lax
from jax.experimental import pallas as pl
from jax.experimental.pallas import tpu as pltpu
from jax.experimental.pallas import tpu_sc as plsc

N = 10000
D = 128
E = 320000

NC = 2
NS = 16
NW = NC * NS
PER_W = E // NW
EC = 88
PER_W_PAD = 10032
ECN = PER_W_PAD // EC
N_PAD = 10112
ROWS_PER_TILE = N_PAD // NS
DEG_CHUNK = 2000

_mesh = plsc.VectorSubcoreMesh(
    core_axis_name="c", subcore_axis_name="s", num_cores=NC, num_subcores=NS
)
_sc_params = pltpu.CompilerParams(needs_layout_passes=False)


@functools.partial(
    pl.kernel,
    out_type=jax.ShapeDtypeStruct((NW * N,), jnp.float32),
    mesh=_mesh,
    scratch_types=[
        pltpu.VMEM((N,), jnp.float32),
        pltpu.VMEM((DEG_CHUNK,), jnp.int32),
    ],
    compiler_params=_sc_params,
)
def _deg_kernel(dst_hbm, out_hbm, hist, idxv):
    cid = lax.axis_index("c")
    sid = lax.axis_index("s")
    wid = sid * NC + cid
    zeros16 = jnp.zeros((16,), jnp.float32)
    ones16 = jnp.ones((16,), jnp.float32)

    def zero_body(i, carry):
        hist[pl.ds(i * 16, 16)] = zeros16
        return carry

    lax.fori_loop(0, N // 16, zero_body, 0)

    def chunk_body(c, carry):
        pltpu.sync_copy(
            dst_hbm.at[pl.ds(wid * PER_W + c * DEG_CHUNK, DEG_CHUNK)], idxv
        )

        def inner(j, carry2):
            idx = idxv[pl.ds(j * 16, 16)]
            plsc.addupdate_scatter(hist, [idx], ones16)
            return carry2

        lax.fori_loop(0, DEG_CHUNK // 16, inner, 0)
        return carry

    lax.fori_loop(0, PER_W // DEG_CHUNK, chunk_body, 0)
    pltpu.sync_copy(hist, out_hbm.at[pl.ds(wid * N, N)])


@functools.partial(
    pl.kernel,
    out_type=jax.ShapeDtypeStruct((NC, N_PAD, D), jnp.float32),
    mesh=_mesh,
    scratch_types=[
        pltpu.VMEM((EC,), jnp.int32),
        pltpu.VMEM((EC,), jnp.int32),
        pltpu.VMEM((EC,), jnp.int32),
        pltpu.VMEM((EC,), jnp.int32),
        pltpu.VMEM((EC, D), jnp.float32),
        pltpu.VMEM((EC, D), jnp.float32),
        pltpu.VMEM_SHARED((N_PAD, D), jnp.float32),
        pltpu.SemaphoreType.DMA,
        pltpu.SemaphoreType.DMA,
        pltpu.SemaphoreType.DMA,
        pltpu.SemaphoreType.DMA,
        pltpu.SemaphoreType.DMA,
        pltpu.SemaphoreType.DMA,
    ],
    compiler_params=_sc_params,
)
def _edge_kernel(
    g_hbm, src_hbm, dst_hbm, zero_hbm, out_hbm,
    srcA, srcB, dstA, dstB, rows0, rows1, acc,
    sem0, sem1, semS0, semS1, semD0, semD1,
):
    cid = lax.axis_index("c")
    sid = lax.axis_index("s")
    wid = sid * NC + cid
    base_row = sid * ROWS_PER_TILE
    ebase = wid * PER_W_PAD

    pltpu.sync_copy(src_hbm.at[pl.ds(ebase, EC)], srcA)
    pltpu.async_copy(g_hbm.at[srcA], rows0, sem0)
    pltpu.sync_copy(src_hbm.at[pl.ds(ebase + EC, EC)], srcB)
    pltpu.async_copy(dst_hbm.at[pl.ds(ebase, EC)], dstA, semD0)
    pltpu.sync_copy(
        zero_hbm.at[pl.ds(base_row, ROWS_PER_TILE)],
        acc.at[pl.ds(base_row, ROWS_PER_TILE)],
    )
    plsc.subcore_barrier()

    def chunk_body(i, carry):
        k0 = 2 * i
        k1 = k0 + 1
        pltpu.make_async_copy(g_hbm.at[srcA], rows0, sem0).wait()
        pltpu.async_copy(g_hbm.at[srcB], rows1, sem1)

        @pl.when(k0 + 2 < ECN)
        def _():
            pltpu.async_copy(src_hbm.at[pl.ds(ebase + (k0 + 2) * EC, EC)], srcA, semS0)

        pltpu.make_async_copy(dst_hbm.at[pl.ds(ebase + k0 * EC, EC)], dstA, semD0).wait()
        pltpu.async_copy(dst_hbm.at[pl.ds(ebase + k1 * EC, EC)], dstB, semD1)
        pltpu.sync_copy(rows0, acc.at[dstA], add=True)
        pltpu.make_async_copy(g_hbm.at[srcB], rows1, sem1).wait()

        @pl.when(k0 + 2 < ECN)
        def _():
            pltpu.make_async_copy(
                src_hbm.at[pl.ds(ebase + (k0 + 2) * EC, EC)], srcA, semS0
            ).wait()
            pltpu.async_copy(g_hbm.at[srcA], rows0, sem0)

        @pl.when(k0 + 3 < ECN)
        def _():
            pltpu.async_copy(src_hbm.at[pl.ds(ebase + (k0 + 3) * EC, EC)], srcB, semS1)

        pltpu.make_async_copy(dst_hbm.at[pl.ds(ebase + k1 * EC, EC)], dstB, semD1).wait()
        pltpu.sync_copy(rows1, acc.at[dstB], add=True)

        @pl.when(k0 + 3 < ECN)
        def _():
            pltpu.make_async_copy(
                src_hbm.at[pl.ds(ebase + (k0 + 3) * EC, EC)], srcB, semS1
            ).wait()

        @pl.when(k0 + 2 < ECN)
        def _():
            pltpu.async_copy(dst_hbm.at[pl.ds(ebase + (k0 + 2) * EC, EC)], dstA, semD0)

        return carry

    lax.fori_loop(0, ECN // 2, chunk_body, 0)
    plsc.subcore_barrier()
    pltpu.sync_copy(
        acc.at[pl.ds(base_row, ROWS_PER_TILE)],
        out_hbm.at[cid, pl.ds(base_row, ROWS_PER_TILE)],
    )


R = 1024


def _dinv_from_hist(hist_blk):
    deg = jnp.sum(hist_blk, axis=0) + 1.0
    return lax.rsqrt(deg)


def _pre_body(hist_ref, x_ref, w1_ref, g1_ref):
    dinv = _dinv_from_hist(hist_ref[...])[:, None]
    h = jnp.dot(x_ref[...], w1_ref[...], preferred_element_type=jnp.float32)
    g1_ref[...] = h * dinv


def _mid_body(p_ref, g1_ref, hist_ref, b1_ref, w2_ref, g2_ref):
    dinv = _dinv_from_hist(hist_ref[...])[:, None]
    h1 = jnp.maximum(dinv * (p_ref[0] + p_ref[1] + g1_ref[...]) + b1_ref[...], 0.0)
    g2_ref[...] = (
        jnp.dot(h1, w2_ref[...], preferred_element_type=jnp.float32) * dinv
    )


def _post_body(q_ref, g2_ref, hist_ref, b2_ref, out_ref):
    dinv = _dinv_from_hist(hist_ref[...])[:, None]
    out_ref[...] = jnp.maximum(
        dinv * (q_ref[0] + q_ref[1] + g2_ref[...]) + b2_ref[...], 0.0
    )


_row_spec = pl.BlockSpec((R, D), lambda i: (i, 0))
_pair_spec = pl.BlockSpec((NC, R, D), lambda i: (0, i, 0))
_hist_spec = pl.BlockSpec((NW, R), lambda i: (0, i))
_w_spec = pl.BlockSpec((D, D), lambda i: (0, 0))
_b_spec = pl.BlockSpec((D,), lambda i: (0,))
_nd_shape = jax.ShapeDtypeStruct((N, D), jnp.float32)
_grid = (pl.cdiv(N, R),)

_pre_call = pl.pallas_call(
    _pre_body,
    grid=_grid,
    in_specs=[_hist_spec, _row_spec, _w_spec],
    out_specs=_row_spec,
    out_shape=_nd_shape,
)

_mid_call = pl.pallas_call(
    _mid_body,
    grid=_grid,
    in_specs=[_pair_spec, _row_spec, _hist_spec, _b_spec, _w_spec],
    out_specs=_row_spec,
    out_shape=_nd_shape,
)

_post_call = pl.pallas_call(
    _post_body,
    grid=_grid,
    in_specs=[_pair_spec, _row_spec, _hist_spec, _b_spec],
    out_specs=_row_spec,
    out_shape=_nd_shape,
)


def kernel(x, edge_index, W1, b1, W2, b2):
    pad = PER_W_PAD - PER_W
    src = jnp.reshape(
        jnp.pad(jnp.reshape(edge_index[0], (NW, PER_W)), ((0, 0), (0, pad))),
        (NW * PER_W_PAD,),
    )
    dst = edge_index[1]
    pad_rows = jnp.broadcast_to(
        (N + jnp.arange(NW, dtype=jnp.int32))[:, None], (NW, pad)
    )
    dst3 = jnp.reshape(
        jnp.concatenate([jnp.reshape(dst, (NW, PER_W)), pad_rows], axis=1),
        (NW * PER_W_PAD,),
    )
    hist = jnp.reshape(_deg_kernel(dst), (NW, N))
    zeros = jnp.zeros((N_PAD, D), jnp.float32)
    g1 = _pre_call(hist, x, W1)
    p = _edge_kernel(g1, src, dst3, zeros)
    g2 = _mid_call(p, g1, hist, b1, W2)
    q = _edge_kernel(g2, src, dst3, zeros)
    return _post_call(q, g2, hist, b2)
